# baseline (speedup 1.0000x reference)
import jax
import jax.numpy as jnp
from jax import lax
from jax.experimental import pallas as pl
from jax.experimental.pallas import tpu as pltpu

NSLICE = 32
NI = 8
BM = 512
HALF = 2048
RS = 128
T_B = 16
T_C = 40


def _wslice(t):
    tb = (t - T_B) // 3
    pos = (t - T_B) % 3
    return jnp.where(
        t < T_B, t, jnp.where(t < T_C, 15 + 2 * tb + pos, NSLICE - 1)
    )


def _xi(t):
    return jnp.where(t < T_B, 0, jnp.where(t < T_C, (t - T_B) // 3, t - T_C))


def _linear_kernel(x_ref, w_ref, b_ref, o_ref, ws_ref):
    t = pl.program_id(0)
    pos = (t - T_B) % 3
    is_fill = (t < T_B) | ((t < T_C) & (pos > 0))

    @pl.when(is_fill)
    def _fill():
        s = _wslice(t)
        ws_ref[pl.ds(s * RS, RS), :] = w_ref[...].astype(jnp.bfloat16)

    @pl.when(jnp.logical_not(is_fill))
    def _compute():
        j = jnp.where(t < T_C, 0, 1)
        xb = x_ref[...].astype(jnp.bfloat16)
        o_ref[...] = lax.dot_general(
            xb, ws_ref[pl.ds(j * HALF, HALF), :], (((1,), (1,)), ((), ())),
            preferred_element_type=jnp.float32,
        ) + b_ref[...]


def kernel(input, weight, bias):
    m, kdim = input.shape
    n, _ = weight.shape
    bias2 = bias.reshape(1, n)
    return pl.pallas_call(
        _linear_kernel,
        grid=(T_C + NI,),
        in_specs=[
            pl.BlockSpec((BM, kdim), lambda t: (_xi(t), 0)),
            pl.BlockSpec((RS, kdim), lambda t: (_wslice(t), 0)),
            pl.BlockSpec((1, HALF), lambda t: (0, jnp.where(t < T_C, 0, 1))),
        ],
        out_specs=pl.BlockSpec(
            (BM, HALF), lambda t: (_xi(t), jnp.where(t < T_C, 0, 1))
        ),
        out_shape=jax.ShapeDtypeStruct((m, n), jnp.float32),
        scratch_shapes=[pltpu.VMEM((n, kdim), jnp.bfloat16)],
        compiler_params=pltpu.CompilerParams(
            dimension_semantics=("arbitrary",),
            vmem_limit_bytes=64 * 1024 * 1024,
        ),
    )(input, weight, bias2)


# speedup vs baseline: 1.2255x; 1.2255x over previous
import jax
import jax.numpy as jnp
from jax import lax
from jax.experimental import pallas as pl
from jax.experimental.pallas import tpu as pltpu

FILL = 16
BM = 256


def _linear_kernel(x_ref, w_ref, b_ref, o_ref, ws_ref):
    t = pl.program_id(0)
    rs = w_ref.shape[0]

    @pl.when(t < FILL)
    def _fill():
        ws_ref[pl.ds(t * rs, rs), :] = w_ref[...].astype(jnp.bfloat16)

    @pl.when(t >= FILL)
    def _compute():
        xb = x_ref[...].astype(jnp.bfloat16)
        o_ref[...] = lax.dot_general(
            xb, ws_ref[...], (((1,), (1,)), ((), ())),
            preferred_element_type=jnp.float32,
        ) + b_ref[...]


def kernel(input, weight, bias):
    m, kdim = input.shape
    n, _ = weight.shape
    bias2 = bias.reshape(1, n)
    nsteps = FILL + m // BM
    return pl.pallas_call(
        _linear_kernel,
        grid=(nsteps,),
        in_specs=[
            pl.BlockSpec((BM, kdim), lambda t: (jnp.maximum(t - FILL, 0), 0)),
            pl.BlockSpec((n // FILL, kdim), lambda t: (jnp.minimum(t, FILL - 1), 0)),
            pl.BlockSpec((1, n), lambda t: (0, 0)),
        ],
        out_specs=pl.BlockSpec((BM, n), lambda t: (jnp.maximum(t - FILL, 0), 0)),
        out_shape=jax.ShapeDtypeStruct((m, n), jnp.float32),
        scratch_shapes=[pltpu.VMEM((n, kdim), jnp.bfloat16)],
        compiler_params=pltpu.CompilerParams(
            dimension_semantics=("arbitrary",),
        ),
    )(input, weight, bias2)
